# baseline (speedup 1.0000x reference)
import functools

import jax
import jax.numpy as jnp
from jax.experimental import pallas as pl
from jax.experimental.pallas import tpu as pltpu

_KNN = 11
_BN = 256


def _voro_kernel(p_ref, sT_ref, out_ref):
    bn = p_ref.shape[1]
    m = sT_ref.shape[2]
    p = p_ref[0]
    sT = sT_ref[0]

    px, py, pz = p[:, 0:1], p[:, 1:2], p[:, 2:3]
    sx, sy, sz = sT[0:1, :], sT[1:2, :], sT[2:3, :]

    dx = px - sx
    dy = py - sy
    dz = pz - sz
    dist = dx * dx + dy * dy + dz * dz

    iota = jax.lax.broadcasted_iota(jnp.int32, (bn, m), 1)
    inf = jnp.float32(jnp.inf)

    d0 = jnp.min(dist, axis=1, keepdims=True)
    i0 = jnp.min(jnp.where(dist == d0, iota, m), axis=1, keepdims=True)
    onehot0 = iota == i0

    cx = jnp.min(jnp.where(onehot0, sx, inf), axis=1, keepdims=True)
    cy = jnp.min(jnp.where(onehot0, sy, inf), axis=1, keepdims=True)
    cz = jnp.min(jnp.where(onehot0, sz, inf), axis=1, keepdims=True)

    ex = cx - sx
    ey = cy - sy
    ez = cz - sz
    dc = ex * ex + ey * ey + ez * ez

    dist = jnp.where(onehot0, inf, dist)

    ans = jnp.full((bn, 1), inf, jnp.float32)
    for _ in range(_KNN - 1):
        mval = jnp.min(dist, axis=1, keepdims=True)
        sel = dist == mval
        dck = jnp.max(jnp.where(sel, dc, -inf), axis=1, keepdims=True)
        num = mval - d0
        ans = jnp.minimum(ans, (num * num) / (4.0 * dck))
        dist = jnp.where(sel, inf, dist)

    out_ref[0] = ans


def _run(points, spoints, interpret=False):
    B, N, _ = points.shape
    M = spoints.shape[1]
    spointsT = jnp.transpose(spoints, (0, 2, 1))
    grid = (B, N // _BN)
    out = pl.pallas_call(
        _voro_kernel,
        grid=grid,
        in_specs=[
            pl.BlockSpec((1, _BN, 3), lambda b, n: (b, n, 0)),
            pl.BlockSpec((1, 3, M), lambda b, n: (b, 0, 0)),
        ],
        out_specs=pl.BlockSpec((1, _BN, 1), lambda b, n: (b, n, 0)),
        out_shape=jax.ShapeDtypeStruct((B, N, 1), jnp.float32),
        compiler_params=pltpu.CompilerParams(
            dimension_semantics=("parallel", "arbitrary"),
        ),
        interpret=interpret,
    )(points, spointsT)
    return out[:, :, 0]


def kernel(points, spoints):
    devs = jax.devices()
    B = points.shape[0]
    nd = 2 if len(devs) >= 2 and B % 2 == 0 else 1
    if nd == 1:
        return _run(points, spoints)
    mesh = jax.sharding.Mesh(devs[:nd], ("d",))
    P = jax.sharding.PartitionSpec
    return jax.shard_map(
        _run,
        mesh=mesh,
        in_specs=(P("d"), P("d")),
        out_specs=P("d"),
        check_vma=False,
    )(points, spoints)


# speedup vs baseline: 1.3922x; 1.3922x over previous
import functools

import jax
import jax.numpy as jnp
from jax.experimental import pallas as pl
from jax.experimental.pallas import tpu as pltpu

_KNN = 11
_BN = 256


def _voro_kernel(p_ref, sT_ref, out_ref):
    bn = p_ref.shape[1]
    m = sT_ref.shape[2]
    p = p_ref[0]
    sT = sT_ref[0]

    px, py, pz = p[:, 0:1], p[:, 1:2], p[:, 2:3]
    sx, sy, sz = sT[0:1, :], sT[1:2, :], sT[2:3, :]

    dx = px - sx
    dy = py - sy
    dz = pz - sz
    dist = dx * dx + dy * dy + dz * dz

    iota = jax.lax.broadcasted_iota(jnp.int32, (bn, m), 1)
    inf = jnp.float32(jnp.inf)

    d0 = jnp.min(dist, axis=1, keepdims=True)
    i0 = jnp.min(jnp.where(dist == d0, iota, m), axis=1, keepdims=True)
    onehot0 = iota == i0

    cx = jnp.min(jnp.where(onehot0, sx, inf), axis=1, keepdims=True)
    cy = jnp.min(jnp.where(onehot0, sy, inf), axis=1, keepdims=True)
    cz = jnp.min(jnp.where(onehot0, sz, inf), axis=1, keepdims=True)

    ex = cx - sx
    ey = cy - sy
    ez = cz - sz
    dc = ex * ex + ey * ey + ez * ez

    m = jnp.min(jnp.where(onehot0, inf, dist), axis=1, keepdims=True)
    for _ in range(_KNN - 2):
        m = jnp.min(jnp.where(dist > m, dist, inf), axis=1, keepdims=True)

    mask = (dist <= m) & jnp.logical_not(onehot0)
    num = dist - d0
    ratio = (num * num) / (4.0 * dc)
    ans = jnp.min(jnp.where(mask, ratio, inf), axis=1, keepdims=True)

    out_ref[0] = ans


def _run(points, spoints, interpret=False):
    B, N, _ = points.shape
    M = spoints.shape[1]
    spointsT = jnp.transpose(spoints, (0, 2, 1))
    grid = (B, N // _BN)
    out = pl.pallas_call(
        _voro_kernel,
        grid=grid,
        in_specs=[
            pl.BlockSpec((1, _BN, 3), lambda b, n: (b, n, 0)),
            pl.BlockSpec((1, 3, M), lambda b, n: (b, 0, 0)),
        ],
        out_specs=pl.BlockSpec((1, _BN, 1), lambda b, n: (b, n, 0)),
        out_shape=jax.ShapeDtypeStruct((B, N, 1), jnp.float32),
        compiler_params=pltpu.CompilerParams(
            dimension_semantics=("parallel", "arbitrary"),
        ),
        interpret=interpret,
    )(points, spointsT)
    return out[:, :, 0]


def kernel(points, spoints):
    return _run(points, spoints)


# speedup vs baseline: 1.4919x; 1.0716x over previous
import functools

import jax
import jax.numpy as jnp
from jax.experimental import pallas as pl
from jax.experimental.pallas import tpu as pltpu

_KNN = 11
_BN = 512


def _voro_kernel(p_ref, sT_ref, out_ref):
    bn = p_ref.shape[1]
    m = sT_ref.shape[2]
    p = p_ref[0]
    sT = sT_ref[0]

    px, py, pz = p[:, 0:1], p[:, 1:2], p[:, 2:3]
    sx, sy, sz = sT[0:1, :], sT[1:2, :], sT[2:3, :]

    dx = px - sx
    dy = py - sy
    dz = pz - sz
    dist = dx * dx + dy * dy + dz * dz

    iota = jax.lax.broadcasted_iota(jnp.int32, (bn, m), 1)
    inf = jnp.float32(jnp.inf)

    d0 = jnp.min(dist, axis=1, keepdims=True)
    i0 = jnp.min(jnp.where(dist == d0, iota, m), axis=1, keepdims=True)
    onehot0 = iota == i0

    cx = jnp.min(jnp.where(onehot0, sx, inf), axis=1, keepdims=True)
    cy = jnp.min(jnp.where(onehot0, sy, inf), axis=1, keepdims=True)
    cz = jnp.min(jnp.where(onehot0, sz, inf), axis=1, keepdims=True)

    ex = cx - sx
    ey = cy - sy
    ez = cz - sz
    dc = ex * ex + ey * ey + ez * ez

    dist = jnp.where(onehot0, inf, dist)

    m = jnp.min(dist, axis=1, keepdims=True)
    for _ in range(_KNN - 2):
        m = jnp.min(jnp.where(dist > m, dist, inf), axis=1, keepdims=True)

    num = dist - d0
    ratio = (num * num) / (4.0 * dc)
    ans = jnp.min(jnp.where(dist <= m, ratio, inf), axis=1, keepdims=True)

    out_ref[0] = ans


def _run(points, spoints, interpret=False):
    B, N, _ = points.shape
    M = spoints.shape[1]
    spointsT = jnp.transpose(spoints, (0, 2, 1))
    grid = (B, N // _BN)
    out = pl.pallas_call(
        _voro_kernel,
        grid=grid,
        in_specs=[
            pl.BlockSpec((1, _BN, 3), lambda b, n: (b, n, 0)),
            pl.BlockSpec((1, 3, M), lambda b, n: (b, 0, 0)),
        ],
        out_specs=pl.BlockSpec((1, _BN, 1), lambda b, n: (b, n, 0)),
        out_shape=jax.ShapeDtypeStruct((B, N, 1), jnp.float32),
        compiler_params=pltpu.CompilerParams(
            dimension_semantics=("parallel", "arbitrary"),
        ),
        interpret=interpret,
    )(points, spointsT)
    return out[:, :, 0]


def kernel(points, spoints):
    return _run(points, spoints)


# speedup vs baseline: 1.5065x; 1.0098x over previous
import functools

import jax
import jax.numpy as jnp
from jax.experimental import pallas as pl
from jax.experimental.pallas import tpu as pltpu

_KNN = 11
_BN = 1024


def _voro_kernel(p_ref, sT_ref, out_ref):
    bn = p_ref.shape[1]
    m = sT_ref.shape[2]
    p = p_ref[0]
    sT = sT_ref[0]

    px, py, pz = p[:, 0:1], p[:, 1:2], p[:, 2:3]
    sx, sy, sz = sT[0:1, :], sT[1:2, :], sT[2:3, :]

    dx = px - sx
    dy = py - sy
    dz = pz - sz
    dist = dx * dx + dy * dy + dz * dz

    iota = jax.lax.broadcasted_iota(jnp.int32, (bn, m), 1)
    inf = jnp.float32(jnp.inf)

    d0 = jnp.min(dist, axis=1, keepdims=True)
    i0 = jnp.min(jnp.where(dist == d0, iota, m), axis=1, keepdims=True)
    onehot0 = iota == i0

    cx = jnp.min(jnp.where(onehot0, sx, inf), axis=1, keepdims=True)
    cy = jnp.min(jnp.where(onehot0, sy, inf), axis=1, keepdims=True)
    cz = jnp.min(jnp.where(onehot0, sz, inf), axis=1, keepdims=True)

    ex = cx - sx
    ey = cy - sy
    ez = cz - sz
    dc = ex * ex + ey * ey + ez * ez

    dist = jnp.where(onehot0, inf, dist)

    m = jnp.min(dist, axis=1, keepdims=True)
    for _ in range(_KNN - 2):
        m = jnp.min(jnp.where(dist > m, dist, inf), axis=1, keepdims=True)

    num = dist - d0
    ratio = (num * num) / (4.0 * dc)
    ans = jnp.min(jnp.where(dist <= m, ratio, inf), axis=1, keepdims=True)

    out_ref[0] = ans


def _run(points, spoints, interpret=False):
    B, N, _ = points.shape
    M = spoints.shape[1]
    spointsT = jnp.transpose(spoints, (0, 2, 1))
    grid = (B, N // _BN)
    out = pl.pallas_call(
        _voro_kernel,
        grid=grid,
        in_specs=[
            pl.BlockSpec((1, _BN, 3), lambda b, n: (b, n, 0)),
            pl.BlockSpec((1, 3, M), lambda b, n: (b, 0, 0)),
        ],
        out_specs=pl.BlockSpec((1, _BN, 1), lambda b, n: (b, n, 0)),
        out_shape=jax.ShapeDtypeStruct((B, N, 1), jnp.float32),
        compiler_params=pltpu.CompilerParams(
            dimension_semantics=("parallel", "arbitrary"),
        ),
        interpret=interpret,
    )(points, spointsT)
    return out[:, :, 0]


def kernel(points, spoints):
    return _run(points, spoints)


# speedup vs baseline: 1.5069x; 1.0003x over previous
import jax
import jax.numpy as jnp
from jax.experimental import pallas as pl
from jax.experimental.pallas import tpu as pltpu

_KNN = 11
_BN = 1024


def _voro_kernel(p_ref, sT_ref, out_ref):
    bn = p_ref.shape[1]
    m = sT_ref.shape[2]
    p = p_ref[0]
    sT = sT_ref[0]

    px, py, pz = p[:, 0:1], p[:, 1:2], p[:, 2:3]
    sx, sy, sz = sT[0:1, :], sT[1:2, :], sT[2:3, :]

    dx = px - sx
    dy = py - sy
    dz = pz - sz
    dist = dx * dx + dy * dy + dz * dz

    iota = jax.lax.broadcasted_iota(jnp.int32, (bn, m), 1)
    inf = jnp.float32(jnp.inf)

    d0 = jnp.min(dist, axis=1, keepdims=True)
    i0 = jnp.min(jnp.where(dist == d0, iota, m), axis=1, keepdims=True)
    onehot0 = iota == i0

    cx = jnp.min(jnp.where(onehot0, sx, inf), axis=1, keepdims=True)
    cy = jnp.min(jnp.where(onehot0, sy, inf), axis=1, keepdims=True)
    cz = jnp.min(jnp.where(onehot0, sz, inf), axis=1, keepdims=True)

    ex = cx - sx
    ey = cy - sy
    ez = cz - sz
    dc = ex * ex + ey * ey + ez * ez

    dist = jnp.where(onehot0, inf, dist)

    m = jnp.min(dist, axis=1, keepdims=True)
    for _ in range(_KNN - 2):
        m = jnp.min(jnp.where(dist > m, dist, inf), axis=1, keepdims=True)

    num = dist - d0
    ratio = (num * num) / (4.0 * dc)
    ans = jnp.min(jnp.where(dist <= m, ratio, inf), axis=1, keepdims=True)

    out_ref[0] = ans


def _run(points, spoints, interpret=False):
    B, N, _ = points.shape
    M = spoints.shape[1]
    spointsT = jnp.transpose(spoints, (0, 2, 1))
    grid = (B, N // _BN)
    out = pl.pallas_call(
        _voro_kernel,
        grid=grid,
        in_specs=[
            pl.BlockSpec((1, _BN, 3), lambda b, n: (b, n, 0)),
            pl.BlockSpec((1, 3, M), lambda b, n: (b, 0, 0)),
        ],
        out_specs=pl.BlockSpec((1, _BN, 1), lambda b, n: (b, n, 0)),
        out_shape=jax.ShapeDtypeStruct((B, N, 1), jnp.float32),
        compiler_params=pltpu.CompilerParams(
            dimension_semantics=("parallel", "arbitrary"),
        ),
        interpret=interpret,
    )(points, spointsT)
    return out[:, :, 0]


def kernel(points, spoints):
    return _run(points, spoints)
